# baseline (speedup 1.0000x reference)
import functools

import numpy as np
import jax
import jax.numpy as jnp
from jax import lax
from jax.experimental import pallas as pl
from jax.experimental.pallas import tpu as pltpu
from jax.experimental.pallas import tpu_sc as plsc

_NL = 3
_NA = 3
_NC = 80
_NCH = _NC + 5
_STRIDES = (8, 16, 32)
_ANCH = (np.array([[[10.0, 13.0], [16.0, 30.0], [33.0, 23.0]],
                   [[30.0, 61.0], [62.0, 45.0], [59.0, 119.0]],
                   [[116.0, 90.0], [156.0, 198.0], [373.0, 326.0]]], np.float32)
         / np.array(_STRIDES, np.float32)[:, None, None])
_BAL = (4.0, 1.0, 0.4)
_G_GIOU, _G_OBJ, _G_CLS = 0.05, 1.0, 0.5
_ANCHOR_T = 4.0
_EPS = 1e-7
_OFF = np.array([[0.0, 0.0], [0.5, 0.0], [0.0, 0.5], [-0.5, 0.0], [0.0, -0.5]],
                np.float32)


def _sigmoid(x):
    return 1.0 / (1.0 + jnp.exp(-x))


def _softplus_terms(x):
    return jnp.maximum(x, 0.0) + jnp.log(1.0 + jnp.exp(-jnp.abs(x)))


def _atan_pos(x):
    inv = x > 1.0
    z = jnp.where(inv, 1.0 / jnp.maximum(x, 1e-30), x)
    t = z / (1.0 + jnp.sqrt(1.0 + z * z))
    t2 = t * t
    p = t * (1.0 + t2 * (-1.0 / 3.0 + t2 * (0.2 + t2 * (-1.0 / 7.0
             + t2 * (1.0 / 9.0 - t2 / 11.0)))))
    a = 2.0 * p
    return jnp.where(inv, (np.pi / 2.0) - a, a)


def _k1_body(tinfo_ref, idx_ref, meta_ref, *, hw):
    ntp = tinfo_ref.shape[1]
    cls_t = tinfo_ref[0:1, :]
    xn = tinfo_ref[1:2, :]
    yn = tinfo_ref[2:3, :]
    wn = tinfo_ref[3:4, :]
    hn = tinfo_ref[4:5, :]
    validf = tinfo_ref[5:6, :]
    bidx = tinfo_ref[6:7, :]

    row = lax.broadcasted_iota(jnp.int32, (16, 1), 0)
    a_r = row % 3
    rowf_ok = (row < 15)
    omode0 = jnp.minimum(row // 3, 4)
    zero = jnp.zeros_like(omode0, jnp.float32)

    def _per_row(vals, sel):
        out = zero + vals[0]
        for q in range(1, len(vals)):
            out = jnp.where(sel == q, vals[q], out)
        return out

    offx = _per_row([_OFF[o][0] for o in range(5)], omode0)
    offy = _per_row([_OFF[o][1] for o in range(5)], omode0)

    for i in range(_NL):
        h, w = hw[i]
        gx = xn * w
        gy = yn * h
        gw = wn * w
        gh = hn * h
        aw = _per_row([float(_ANCH[i][q][0]) for q in range(3)], a_r)
        ah = _per_row([float(_ANCH[i][q][1]) for q in range(3)], a_r)

        rw = gw / aw
        rh = gh / ah
        anc_ok = jnp.maximum(jnp.maximum(rw, 1.0 / rw),
                             jnp.maximum(rh, 1.0 / rh)) < _ANCHOR_T

        fx = gx - jnp.floor(gx)
        fy = gy - jnp.floor(gy)
        gxi = w - gx
        gyi = h - gy
        fxi = gxi - jnp.floor(gxi)
        fyi = gyi - jnp.floor(gyi)
        c1 = ((fx < 0.5) & (gx > 1.0)).astype(jnp.float32)
        c2 = ((fy < 0.5) & (gy > 1.0)).astype(jnp.float32)
        c3 = ((fxi < 0.5) & (gxi > 1.0)).astype(jnp.float32)
        c4 = ((fyi < 0.5) & (gyi > 1.0)).astype(jnp.float32)
        omode = row // 3
        w0 = (omode == 0).astype(jnp.float32)
        w1 = (omode == 1).astype(jnp.float32)
        w2 = (omode == 2).astype(jnp.float32)
        w3 = (omode == 3).astype(jnp.float32)
        w4 = (omode == 4).astype(jnp.float32)
        omf = w0 + w1 * c1 + w2 * c2 + w3 * c3 + w4 * c4

        mf = (omf * anc_ok.astype(jnp.float32) * validf
              * rowf_ok.astype(jnp.float32))
        m = mf > 0.5

        sx = gx - offx
        sy = gy - offy
        gi = sx.astype(jnp.int32)
        gj = sy.astype(jnp.int32)
        tbx = gx - gi.astype(jnp.float32)
        tby = gy - gj.astype(jnp.float32)

        b_i = bidx.astype(jnp.int32)
        lin = ((b_i * _NA + a_r) * h + gj) * w + gi
        idx_ref[i] = jnp.where(m, lin, 0)
        meta_ref[i, 0] = jnp.broadcast_to(mf, (16, ntp))
        meta_ref[i, 1] = jnp.broadcast_to(tbx, (16, ntp))
        meta_ref[i, 2] = jnp.broadcast_to(tby, (16, ntp))
        meta_ref[i, 3] = jnp.broadcast_to(gw, (16, ntp))
        meta_ref[i, 4] = jnp.broadcast_to(gh, (16, ntp))
        meta_ref[i, 5] = jnp.broadcast_to(cls_t, (16, ntp))
        meta_ref[i, 6] = jnp.broadcast_to(aw, (16, ntp))
        meta_ref[i, 7] = jnp.broadcast_to(ah, (16, ntp))


def _k2_body(p0_ref, p1_ref, p2_ref, q0_ref, q1_ref, q2_ref, dsum_ref):
    k = pl.program_id(0)

    @pl.when(k == 0)
    def _init():
        dsum_ref[...] = jnp.zeros_like(dsum_ref)

    lane = lax.broadcasted_iota(jnp.int32, (1, 128), 1)
    part = []
    for p_ref, q_ref in ((p0_ref, q0_ref), (p1_ref, q1_ref), (p2_ref, q2_ref)):
        x = p_ref[...]
        part.append(jnp.sum(_softplus_terms(x[:, 4:5])))
        pad = jnp.zeros((x.shape[0], 128 - _NCH), jnp.float32)
        q_ref[...] = jnp.concatenate((x, pad), axis=1)
    dsum_ref[...] = dsum_ref[...] + (jnp.where(lane == 0, part[0], 0.0)
                                     + jnp.where(lane == 1, part[1], 0.0)
                                     + jnp.where(lane == 2, part[2], 0.0))


def _sc_gather(idx0, idx1, idx2, q0, q1, q2):
    nwork, nchunk, _ = idx0.shape
    upw = nchunk * 128
    n = nwork * upw
    mesh = plsc.VectorSubcoreMesh(core_axis_name="c", subcore_axis_name="s")
    row_t = jax.ShapeDtypeStruct((n, 128), jnp.float32)

    @functools.partial(
        pl.kernel,
        out_type=(row_t, row_t, row_t),
        mesh=mesh,
        scratch_types=[
            pltpu.VMEM((nchunk, 128), jnp.int32),
            pltpu.VMEM((upw, 128), jnp.float32),
            pltpu.SemaphoreType.DMA,
        ],
    )
    def k(i0, i1, i2, t0, t1, t2, o0, o1, o2, idx_v, rows_v, sem):
        wid = lax.axis_index("s") * 2 + lax.axis_index("c")
        for ih, tab, out in ((i0, t0, o0), (i1, t1, o1), (i2, t2, o2)):
            pltpu.sync_copy(ih.at[wid], idx_v)
            descs = []
            for c in range(nchunk):
                descs.append(pltpu.async_copy(
                    tab.at[idx_v.at[c]],
                    rows_v.at[pl.ds(c * 128, 128), :], sem))
            for d in descs:
                d.wait()
            pltpu.sync_copy(rows_v, out.at[pl.ds(wid * upw, upw)])

    return k(idx0, idx1, idx2, q0, q1, q2)


def _k4_body(ps0_ref, ps1_ref, ps2_ref, meta_ref, dsum_ref, out_ref, acc_ref,
             *, nsteps, npix, bs):
    j = pl.program_id(0)

    @pl.when(j == 0)
    def _init():
        acc_ref[...] = jnp.zeros_like(acc_ref)

    lane = lax.broadcasted_iota(jnp.int32, (1, 128), 1)
    for i, ps_ref in enumerate((ps0_ref, ps1_ref, ps2_ref)):
        ps = ps_ref[...]
        mt = meta_ref[i]
        m = mt[:, 0:1]
        tbx, tby = mt[:, 1:2], mt[:, 2:3]
        tbw, tbh = mt[:, 3:4], mt[:, 4:5]
        cls_t = mt[:, 5:6]
        aw, ah = mt[:, 6:7], mt[:, 7:8]

        s = _sigmoid(ps[:, 0:4])
        px = s[:, 0:1] * 2.0 - 0.5
        py = s[:, 1:2] * 2.0 - 0.5
        pw = (s[:, 2:3] * 2.0) ** 2 * aw
        ph = (s[:, 3:4] * 2.0) ** 2 * ah

        b1x1, b1x2 = px - pw * 0.5, px + pw * 0.5
        b1y1, b1y2 = py - ph * 0.5, py + ph * 0.5
        b2x1, b2x2 = tbx - tbw * 0.5, tbx + tbw * 0.5
        b2y1, b2y2 = tby - tbh * 0.5, tby + tbh * 0.5
        inter = (jnp.maximum(jnp.minimum(b1x2, b2x2) - jnp.maximum(b1x1, b2x1), 0.0)
                 * jnp.maximum(jnp.minimum(b1y2, b2y2) - jnp.maximum(b1y1, b2y1), 0.0))
        union = pw * ph + tbw * tbh - inter + _EPS
        iou = inter / union
        cw = jnp.maximum(b1x2, b2x2) - jnp.minimum(b1x1, b2x1)
        ch = jnp.maximum(b1y2, b2y2) - jnp.minimum(b1y1, b2y1)
        c2 = cw * cw + ch * ch + _EPS
        rho2 = ((b2x1 + b2x2 - b1x1 - b1x2) ** 2
                + (b2y1 + b2y2 - b1y1 - b1y2) ** 2) * 0.25
        v = ((4.0 / np.pi ** 2)
             * (_atan_pos(tbw / (tbh + _EPS)) - _atan_pos(pw / (ph + _EPS))) ** 2)
        alpha = v / (v - iou + (1.0 + _EPS))
        ciou = iou - (rho2 / c2 + v * alpha)

        lbox_num = jnp.sum((1.0 - ciou) * m)
        x4 = ps[:, 4:5]
        s_obj = jnp.sum(m * x4 * jnp.maximum(ciou, 0.0))
        cnt = jnp.sum(m)

        xs = ps[:, 5:_NCH]
        sp_sum = jnp.sum(_softplus_terms(xs), axis=1, keepdims=True)
        cid = lax.broadcasted_iota(jnp.int32, xs.shape, 1)
        pick = jnp.sum(jnp.where(cid == cls_t.astype(jnp.int32), xs, 0.0),
                       axis=1, keepdims=True)
        cls_num = jnp.sum(m * (sp_sum - pick))

        contrib = (jnp.where(lane == 0, lbox_num, 0.0)
                   + jnp.where(lane == 1, cls_num, 0.0)
                   + jnp.where(lane == 2, s_obj, 0.0)
                   + jnp.where(lane == 3, cnt, 0.0))
        acc_ref[i:i + 1, :] = acc_ref[i:i + 1, :] + contrib

    @pl.when(j == nsteps - 1)
    def _final():
        lbox = jnp.float32(0.0)
        lcls = jnp.float32(0.0)
        lobj = jnp.float32(0.0)
        for i in range(_NL):
            lbox_num = acc_ref[i, 0]
            cls_num = acc_ref[i, 1]
            s_obj = acc_ref[i, 2]
            cnt = acc_ref[i, 3]
            denom = jnp.maximum(cnt, 1.0)
            lbox = lbox + jnp.where(cnt > 0, lbox_num / denom, 0.0)
            lcls = lcls + jnp.where(cnt > 0, cls_num / (denom * _NC), 0.0)
            dense = dsum_ref[0, i]
            lobj = lobj + ((dense - s_obj) / npix[i]) * _BAL[i]
        lbox = lbox * _G_GIOU
        lobj = lobj * _G_OBJ
        lcls = lcls * _G_CLS
        loss = (lbox + lobj + lcls) * bs
        out_ref[0:1, :] = jnp.where(lane == 0, loss, 0.0)
        out_ref[1:2, :] = (jnp.where(lane == 0, lbox, 0.0)
                           + jnp.where(lane == 1, lobj, 0.0)
                           + jnp.where(lane == 2, lcls, 0.0))


def kernel(p0, p1, p2, raw_targets, labels_length, img_size):
    b, mb, _ = raw_targets.shape
    mpad = 64
    ntp = b * mpad
    nupd = 16 * ntp
    hw = tuple((p.shape[2], p.shape[3]) for p in (p0, p1, p2))
    npix = tuple(p.shape[0] * p.shape[1] * p.shape[2] * p.shape[3]
                 for p in (p0, p1, p2))

    rt = jnp.pad(raw_targets, ((0, 0), (0, mpad - mb), (0, 0)))
    isz = jnp.asarray(img_size, jnp.float32)
    validf = (jnp.arange(mpad)[None, :] < labels_length[:, None]).astype(jnp.float32)
    bidxf = jnp.broadcast_to(jnp.arange(b, dtype=jnp.float32)[:, None], (b, mpad))
    tinfo = jnp.stack([
        rt[:, :, 0].reshape(-1),
        (rt[:, :, 1] / isz).reshape(-1),
        (rt[:, :, 2] / isz).reshape(-1),
        (rt[:, :, 3] / isz).reshape(-1),
        (rt[:, :, 4] / isz).reshape(-1),
        validf.reshape(-1),
        bidxf.reshape(-1),
        jnp.zeros((ntp,), jnp.float32),
    ])

    idx, meta = pl.pallas_call(
        functools.partial(_k1_body, hw=hw),
        out_shape=(jax.ShapeDtypeStruct((_NL, 16, ntp), jnp.int32),
                   jax.ShapeDtypeStruct((_NL, 8, 16, ntp), jnp.float32)),
    )(tinfo)

    t0 = p0.reshape(npix[0], _NCH)
    t1 = p1.reshape(npix[1], _NCH)
    t2 = p2.reshape(npix[2], _NCH)
    nsteps = 40
    blk0, blk1, blk2 = npix[0] // nsteps, npix[1] // nsteps, npix[2] // nsteps
    q0, q1, q2, dsum = pl.pallas_call(
        _k2_body,
        grid=(nsteps,),
        in_specs=[
            pl.BlockSpec((blk0, _NCH), lambda k: (k, 0)),
            pl.BlockSpec((blk1, _NCH), lambda k: (k, 0)),
            pl.BlockSpec((blk2, _NCH), lambda k: (k, 0)),
        ],
        out_specs=[
            pl.BlockSpec((blk0, 128), lambda k: (k, 0)),
            pl.BlockSpec((blk1, 128), lambda k: (k, 0)),
            pl.BlockSpec((blk2, 128), lambda k: (k, 0)),
            pl.BlockSpec((1, 128), lambda k: (0, 0)),
        ],
        out_shape=(jax.ShapeDtypeStruct((npix[0], 128), jnp.float32),
                   jax.ShapeDtypeStruct((npix[1], 128), jnp.float32),
                   jax.ShapeDtypeStruct((npix[2], 128), jnp.float32),
                   jax.ShapeDtypeStruct((1, 128), jnp.float32)),
    )(t0, t1, t2)

    idx_w = idx.reshape(_NL, 32, nupd // (32 * 128), 128)
    idx_w = jnp.zeros_like(idx_w)
    ps0, ps1, ps2 = _sc_gather(idx_w[0], idx_w[1], idx_w[2], q0, q1, q2)

    meta_u = jnp.transpose(meta.reshape(_NL, 8, nupd), (0, 2, 1))
    blk = 2048
    nblk = nupd // blk
    out = pl.pallas_call(
        functools.partial(_k4_body, nsteps=nblk, npix=npix, bs=float(b)),
        grid=(nblk,),
        in_specs=[
            pl.BlockSpec((blk, 128), lambda j: (j, 0)),
            pl.BlockSpec((blk, 128), lambda j: (j, 0)),
            pl.BlockSpec((blk, 128), lambda j: (j, 0)),
            pl.BlockSpec((_NL, blk, 8), lambda j: (0, j, 0)),
            pl.BlockSpec((1, 128), lambda j: (0, 0)),
        ],
        out_specs=pl.BlockSpec((8, 128), lambda j: (0, 0)),
        out_shape=jax.ShapeDtypeStruct((8, 128), jnp.float32),
        scratch_shapes=[pltpu.VMEM((8, 128), jnp.float32)],
    )(ps0, ps1, ps2, meta_u, dsum)

    loss = out[0, 0:1]
    stack = out[1, 0:3]
    return (loss, stack)


# speedup vs baseline: 4.8813x; 4.8813x over previous
import functools

import numpy as np
import jax
import jax.numpy as jnp
from jax import lax
from jax.experimental import pallas as pl
from jax.experimental.pallas import tpu as pltpu
from jax.experimental.pallas import tpu_sc as plsc

_NL = 3
_NA = 3
_NC = 80
_NCH = _NC + 5
_STRIDES = (8, 16, 32)
_ANCH = (np.array([[[10.0, 13.0], [16.0, 30.0], [33.0, 23.0]],
                   [[30.0, 61.0], [62.0, 45.0], [59.0, 119.0]],
                   [[116.0, 90.0], [156.0, 198.0], [373.0, 326.0]]], np.float32)
         / np.array(_STRIDES, np.float32)[:, None, None])
_BAL = (4.0, 1.0, 0.4)
_G_GIOU, _G_OBJ, _G_CLS = 0.05, 1.0, 0.5
_ANCHOR_T = 4.0
_EPS = 1e-7
_OFF = np.array([[0.0, 0.0], [0.5, 0.0], [0.0, 0.5], [-0.5, 0.0], [0.0, -0.5]],
                np.float32)


def _sigmoid(x):
    return 1.0 / (1.0 + jnp.exp(-x))


def _softplus_terms(x):
    return jnp.maximum(x, 0.0) + jnp.log(1.0 + jnp.exp(-jnp.abs(x)))


def _atan_pos(x):
    inv = x > 1.0
    z = jnp.where(inv, 1.0 / jnp.maximum(x, 1e-30), x)
    t = z / (1.0 + jnp.sqrt(1.0 + z * z))
    t2 = t * t
    p = t * (1.0 + t2 * (-1.0 / 3.0 + t2 * (0.2 + t2 * (-1.0 / 7.0
             + t2 * (1.0 / 9.0 - t2 / 11.0)))))
    a = 2.0 * p
    return jnp.where(inv, (np.pi / 2.0) - a, a)


def _k1_body(tinfo_ref, idx_ref, meta_ref, *, hw):
    ntp = tinfo_ref.shape[1]
    cls_t = tinfo_ref[0:1, :]
    xn = tinfo_ref[1:2, :]
    yn = tinfo_ref[2:3, :]
    wn = tinfo_ref[3:4, :]
    hn = tinfo_ref[4:5, :]
    validf = tinfo_ref[5:6, :]
    bidx = tinfo_ref[6:7, :]

    row = lax.broadcasted_iota(jnp.int32, (16, 1), 0)
    a_r = row % 3
    rowf_ok = (row < 15)
    omode0 = jnp.minimum(row // 3, 4)
    zero = jnp.zeros_like(omode0, jnp.float32)

    def _per_row(vals, sel):
        out = zero + vals[0]
        for q in range(1, len(vals)):
            out = jnp.where(sel == q, vals[q], out)
        return out

    offx = _per_row([_OFF[o][0] for o in range(5)], omode0)
    offy = _per_row([_OFF[o][1] for o in range(5)], omode0)

    for i in range(_NL):
        h, w = hw[i]
        gx = xn * w
        gy = yn * h
        gw = wn * w
        gh = hn * h
        aw = _per_row([float(_ANCH[i][q][0]) for q in range(3)], a_r)
        ah = _per_row([float(_ANCH[i][q][1]) for q in range(3)], a_r)

        rw = gw / aw
        rh = gh / ah
        anc_ok = jnp.maximum(jnp.maximum(rw, 1.0 / rw),
                             jnp.maximum(rh, 1.0 / rh)) < _ANCHOR_T

        fx = gx - jnp.floor(gx)
        fy = gy - jnp.floor(gy)
        gxi = w - gx
        gyi = h - gy
        fxi = gxi - jnp.floor(gxi)
        fyi = gyi - jnp.floor(gyi)
        c1 = ((fx < 0.5) & (gx > 1.0)).astype(jnp.float32)
        c2 = ((fy < 0.5) & (gy > 1.0)).astype(jnp.float32)
        c3 = ((fxi < 0.5) & (gxi > 1.0)).astype(jnp.float32)
        c4 = ((fyi < 0.5) & (gyi > 1.0)).astype(jnp.float32)
        omode = row // 3
        w0 = (omode == 0).astype(jnp.float32)
        w1 = (omode == 1).astype(jnp.float32)
        w2 = (omode == 2).astype(jnp.float32)
        w3 = (omode == 3).astype(jnp.float32)
        w4 = (omode == 4).astype(jnp.float32)
        omf = w0 + w1 * c1 + w2 * c2 + w3 * c3 + w4 * c4

        mf = (omf * anc_ok.astype(jnp.float32) * validf
              * rowf_ok.astype(jnp.float32))
        m = mf > 0.5

        sx = gx - offx
        sy = gy - offy
        gi = sx.astype(jnp.int32)
        gj = sy.astype(jnp.int32)
        tbx = gx - gi.astype(jnp.float32)
        tby = gy - gj.astype(jnp.float32)

        b_i = bidx.astype(jnp.int32)
        lin = ((b_i * _NA + a_r) * h + gj) * w + gi
        col = lax.broadcasted_iota(jnp.int32, (16, ntp), 1)
        spread = row * ntp + col
        idx_ref[i] = jnp.where(m, lin, spread)
        meta_ref[i, 0] = jnp.broadcast_to(mf, (16, ntp))
        meta_ref[i, 1] = jnp.broadcast_to(tbx, (16, ntp))
        meta_ref[i, 2] = jnp.broadcast_to(tby, (16, ntp))
        meta_ref[i, 3] = jnp.broadcast_to(gw, (16, ntp))
        meta_ref[i, 4] = jnp.broadcast_to(gh, (16, ntp))
        meta_ref[i, 5] = jnp.broadcast_to(cls_t, (16, ntp))
        meta_ref[i, 6] = jnp.broadcast_to(aw, (16, ntp))
        meta_ref[i, 7] = jnp.broadcast_to(ah, (16, ntp))


def _k2_body(p0_ref, p1_ref, p2_ref, q0_ref, q1_ref, q2_ref, dsum_ref):
    k = pl.program_id(0)

    @pl.when(k == 0)
    def _init():
        dsum_ref[...] = jnp.zeros_like(dsum_ref)

    lane = lax.broadcasted_iota(jnp.int32, (1, 128), 1)
    part = []
    for p_ref, q_ref in ((p0_ref, q0_ref), (p1_ref, q1_ref), (p2_ref, q2_ref)):
        x = p_ref[...]
        part.append(jnp.sum(_softplus_terms(x[:, 4:5])))
        pad = jnp.zeros((x.shape[0], 128 - _NCH), jnp.float32)
        q_ref[...] = jnp.concatenate((x, pad), axis=1)
    dsum_ref[...] = dsum_ref[...] + (jnp.where(lane == 0, part[0], 0.0)
                                     + jnp.where(lane == 1, part[1], 0.0)
                                     + jnp.where(lane == 2, part[2], 0.0))


def _sc_gather(idx0, idx1, idx2, q0, q1, q2):
    nwork, nchunk, _ = idx0.shape
    upw = nchunk * 128
    n = nwork * upw
    mesh = plsc.VectorSubcoreMesh(core_axis_name="c", subcore_axis_name="s")
    row_t = jax.ShapeDtypeStruct((n, 128), jnp.float32)

    @functools.partial(
        pl.kernel,
        out_type=(row_t, row_t, row_t),
        mesh=mesh,
        scratch_types=[
            pltpu.VMEM((nchunk, 128), jnp.int32),
            pltpu.VMEM((upw, 128), jnp.float32),
            pltpu.SemaphoreType.DMA,
        ],
    )
    def k(i0, i1, i2, t0, t1, t2, o0, o1, o2, idx_v, rows_v, sem):
        wid = lax.axis_index("s") * 2 + lax.axis_index("c")
        for ih, tab, out in ((i0, t0, o0), (i1, t1, o1), (i2, t2, o2)):
            pltpu.sync_copy(ih.at[wid], idx_v)
            descs = []
            for c in range(nchunk):
                descs.append(pltpu.async_copy(
                    tab.at[idx_v.at[c]],
                    rows_v.at[pl.ds(c * 128, 128), :], sem))
            for d in descs:
                d.wait()
            pltpu.sync_copy(rows_v, out.at[pl.ds(wid * upw, upw)])

    return k(idx0, idx1, idx2, q0, q1, q2)


def _k4_body(ps0_ref, ps1_ref, ps2_ref, meta_ref, dsum_ref, out_ref, acc_ref,
             *, nsteps, npix, bs):
    j = pl.program_id(0)

    @pl.when(j == 0)
    def _init():
        acc_ref[...] = jnp.zeros_like(acc_ref)

    lane = lax.broadcasted_iota(jnp.int32, (1, 128), 1)
    for i, ps_ref in enumerate((ps0_ref, ps1_ref, ps2_ref)):
        ps = ps_ref[...].astype(jnp.float32)
        mt = meta_ref[i]
        m = mt[:, 0:1]
        tbx, tby = mt[:, 1:2], mt[:, 2:3]
        tbw, tbh = mt[:, 3:4], mt[:, 4:5]
        cls_t = mt[:, 5:6]
        aw, ah = mt[:, 6:7], mt[:, 7:8]

        s = _sigmoid(ps[:, 0:4])
        px = s[:, 0:1] * 2.0 - 0.5
        py = s[:, 1:2] * 2.0 - 0.5
        pw = (s[:, 2:3] * 2.0) ** 2 * aw
        ph = (s[:, 3:4] * 2.0) ** 2 * ah

        b1x1, b1x2 = px - pw * 0.5, px + pw * 0.5
        b1y1, b1y2 = py - ph * 0.5, py + ph * 0.5
        b2x1, b2x2 = tbx - tbw * 0.5, tbx + tbw * 0.5
        b2y1, b2y2 = tby - tbh * 0.5, tby + tbh * 0.5
        inter = (jnp.maximum(jnp.minimum(b1x2, b2x2) - jnp.maximum(b1x1, b2x1), 0.0)
                 * jnp.maximum(jnp.minimum(b1y2, b2y2) - jnp.maximum(b1y1, b2y1), 0.0))
        union = pw * ph + tbw * tbh - inter + _EPS
        iou = inter / union
        cw = jnp.maximum(b1x2, b2x2) - jnp.minimum(b1x1, b2x1)
        ch = jnp.maximum(b1y2, b2y2) - jnp.minimum(b1y1, b2y1)
        c2 = cw * cw + ch * ch + _EPS
        rho2 = ((b2x1 + b2x2 - b1x1 - b1x2) ** 2
                + (b2y1 + b2y2 - b1y1 - b1y2) ** 2) * 0.25
        v = ((4.0 / np.pi ** 2)
             * (_atan_pos(tbw / (tbh + _EPS)) - _atan_pos(pw / (ph + _EPS))) ** 2)
        alpha = v / (v - iou + (1.0 + _EPS))
        ciou = iou - (rho2 / c2 + v * alpha)

        lbox_num = jnp.sum((1.0 - ciou) * m)
        x4 = ps[:, 4:5]
        s_obj = jnp.sum(m * x4 * jnp.maximum(ciou, 0.0))
        cnt = jnp.sum(m)

        xs = ps[:, 5:_NCH]
        sp_sum = jnp.sum(_softplus_terms(xs), axis=1, keepdims=True)
        cid = lax.broadcasted_iota(jnp.int32, xs.shape, 1)
        pick = jnp.sum(jnp.where(cid == cls_t.astype(jnp.int32), xs, 0.0),
                       axis=1, keepdims=True)
        cls_num = jnp.sum(m * (sp_sum - pick))

        contrib = (jnp.where(lane == 0, lbox_num, 0.0)
                   + jnp.where(lane == 1, cls_num, 0.0)
                   + jnp.where(lane == 2, s_obj, 0.0)
                   + jnp.where(lane == 3, cnt, 0.0))
        acc_ref[i:i + 1, :] = acc_ref[i:i + 1, :] + contrib

    @pl.when(j == nsteps - 1)
    def _final():
        lbox = jnp.float32(0.0)
        lcls = jnp.float32(0.0)
        lobj = jnp.float32(0.0)
        for i in range(_NL):
            lbox_num = acc_ref[i, 0]
            cls_num = acc_ref[i, 1]
            s_obj = acc_ref[i, 2]
            cnt = acc_ref[i, 3]
            denom = jnp.maximum(cnt, 1.0)
            lbox = lbox + jnp.where(cnt > 0, lbox_num / denom, 0.0)
            lcls = lcls + jnp.where(cnt > 0, cls_num / (denom * _NC), 0.0)
            dense = dsum_ref[0, i]
            lobj = lobj + ((dense - s_obj) / npix[i]) * _BAL[i]
        lbox = lbox * _G_GIOU
        lobj = lobj * _G_OBJ
        lcls = lcls * _G_CLS
        loss = (lbox + lobj + lcls) * bs
        out_ref[0:1, :] = jnp.where(lane == 0, loss, 0.0)
        out_ref[1:2, :] = (jnp.where(lane == 0, lbox, 0.0)
                           + jnp.where(lane == 1, lobj, 0.0)
                           + jnp.where(lane == 2, lcls, 0.0))


def kernel(p0, p1, p2, raw_targets, labels_length, img_size):
    b, mb, _ = raw_targets.shape
    mpad = 64
    ntp = b * mpad
    nupd = 16 * ntp
    hw = tuple((p.shape[2], p.shape[3]) for p in (p0, p1, p2))
    npix = tuple(p.shape[0] * p.shape[1] * p.shape[2] * p.shape[3]
                 for p in (p0, p1, p2))

    rt = jnp.pad(raw_targets, ((0, 0), (0, mpad - mb), (0, 0)))
    isz = jnp.asarray(img_size, jnp.float32)
    validf = (jnp.arange(mpad)[None, :] < labels_length[:, None]).astype(jnp.float32)
    bidxf = jnp.broadcast_to(jnp.arange(b, dtype=jnp.float32)[:, None], (b, mpad))
    tinfo = jnp.stack([
        rt[:, :, 0].reshape(-1),
        (rt[:, :, 1] / isz).reshape(-1),
        (rt[:, :, 2] / isz).reshape(-1),
        (rt[:, :, 3] / isz).reshape(-1),
        (rt[:, :, 4] / isz).reshape(-1),
        validf.reshape(-1),
        bidxf.reshape(-1),
        jnp.zeros((ntp,), jnp.float32),
    ])

    idx, meta = pl.pallas_call(
        functools.partial(_k1_body, hw=hw),
        out_shape=(jax.ShapeDtypeStruct((_NL, 16, ntp), jnp.int32),
                   jax.ShapeDtypeStruct((_NL, 8, 16, ntp), jnp.float32)),
    )(tinfo)

    t0 = p0.reshape(npix[0], _NCH)
    t1 = p1.reshape(npix[1], _NCH)
    t2 = p2.reshape(npix[2], _NCH)
    nsteps = 40
    blk0, blk1, blk2 = npix[0] // nsteps, npix[1] // nsteps, npix[2] // nsteps
    q0, q1, q2, dsum = pl.pallas_call(
        _k2_body,
        grid=(nsteps,),
        in_specs=[
            pl.BlockSpec((blk0, _NCH), lambda k: (k, 0)),
            pl.BlockSpec((blk1, _NCH), lambda k: (k, 0)),
            pl.BlockSpec((blk2, _NCH), lambda k: (k, 0)),
        ],
        out_specs=[
            pl.BlockSpec((blk0, 128), lambda k: (k, 0)),
            pl.BlockSpec((blk1, 128), lambda k: (k, 0)),
            pl.BlockSpec((blk2, 128), lambda k: (k, 0)),
            pl.BlockSpec((1, 128), lambda k: (0, 0)),
        ],
        out_shape=(jax.ShapeDtypeStruct((npix[0], 128), jnp.float32),
                   jax.ShapeDtypeStruct((npix[1], 128), jnp.float32),
                   jax.ShapeDtypeStruct((npix[2], 128), jnp.float32),
                   jax.ShapeDtypeStruct((1, 128), jnp.float32)),
    )(t0, t1, t2)

    idx_w = idx.reshape(_NL, 32, nupd // (32 * 128), 128)
    ps0, ps1, ps2 = _sc_gather(idx_w[0], idx_w[1], idx_w[2], q0, q1, q2)

    meta_u = jnp.transpose(meta.reshape(_NL, 8, nupd), (0, 2, 1))
    blk = 2048
    nblk = nupd // blk
    out = pl.pallas_call(
        functools.partial(_k4_body, nsteps=nblk, npix=npix, bs=float(b)),
        grid=(nblk,),
        in_specs=[
            pl.BlockSpec((blk, 128), lambda j: (j, 0)),
            pl.BlockSpec((blk, 128), lambda j: (j, 0)),
            pl.BlockSpec((blk, 128), lambda j: (j, 0)),
            pl.BlockSpec((_NL, blk, 8), lambda j: (0, j, 0)),
            pl.BlockSpec((1, 128), lambda j: (0, 0)),
        ],
        out_specs=pl.BlockSpec((8, 128), lambda j: (0, 0)),
        out_shape=jax.ShapeDtypeStruct((8, 128), jnp.float32),
        scratch_shapes=[pltpu.VMEM((8, 128), jnp.float32)],
    )(ps0, ps1, ps2, meta_u, dsum)

    loss = out[0, 0:1]
    stack = out[1, 0:3]
    return (loss, stack)


# speedup vs baseline: 4.9747x; 1.0191x over previous
import functools

import numpy as np
import jax
import jax.numpy as jnp
from jax import lax
from jax.experimental import pallas as pl
from jax.experimental.pallas import tpu as pltpu
from jax.experimental.pallas import tpu_sc as plsc

_NL = 3
_NA = 3
_NC = 80
_NCH = _NC + 5
_STRIDES = (8, 16, 32)
_ANCH = (np.array([[[10.0, 13.0], [16.0, 30.0], [33.0, 23.0]],
                   [[30.0, 61.0], [62.0, 45.0], [59.0, 119.0]],
                   [[116.0, 90.0], [156.0, 198.0], [373.0, 326.0]]], np.float32)
         / np.array(_STRIDES, np.float32)[:, None, None])
_BAL = (4.0, 1.0, 0.4)
_G_GIOU, _G_OBJ, _G_CLS = 0.05, 1.0, 0.5
_ANCHOR_T = 4.0
_EPS = 1e-7
_OFF = np.array([[0.0, 0.0], [0.5, 0.0], [0.0, 0.5], [-0.5, 0.0], [0.0, -0.5]],
                np.float32)


def _sigmoid(x):
    return 1.0 / (1.0 + jnp.exp(-x))


def _softplus_terms(x):
    return jnp.maximum(x, 0.0) + jnp.log(1.0 + jnp.exp(-jnp.abs(x)))


def _atan_pos(x):
    inv = x > 1.0
    z = jnp.where(inv, 1.0 / jnp.maximum(x, 1e-30), x)
    t = z / (1.0 + jnp.sqrt(1.0 + z * z))
    t2 = t * t
    p = t * (1.0 + t2 * (-1.0 / 3.0 + t2 * (0.2 + t2 * (-1.0 / 7.0
             + t2 * (1.0 / 9.0 - t2 / 11.0)))))
    a = 2.0 * p
    return jnp.where(inv, (np.pi / 2.0) - a, a)


def _k1_body(tinfo_ref, idx_ref, meta_ref, *, hw):
    ntp = tinfo_ref.shape[1]
    cls_t = tinfo_ref[0:1, :]
    xn = tinfo_ref[1:2, :]
    yn = tinfo_ref[2:3, :]
    wn = tinfo_ref[3:4, :]
    hn = tinfo_ref[4:5, :]
    validf = tinfo_ref[5:6, :]
    bidx = tinfo_ref[6:7, :]

    row = lax.broadcasted_iota(jnp.int32, (16, 1), 0)
    a_r = row % 3
    rowf_ok = (row < 15)
    omode0 = jnp.minimum(row // 3, 4)
    zero = jnp.zeros_like(omode0, jnp.float32)

    def _per_row(vals, sel):
        out = zero + vals[0]
        for q in range(1, len(vals)):
            out = jnp.where(sel == q, vals[q], out)
        return out

    offx = _per_row([_OFF[o][0] for o in range(5)], omode0)
    offy = _per_row([_OFF[o][1] for o in range(5)], omode0)

    for i in range(_NL):
        h, w = hw[i]
        gx = xn * w
        gy = yn * h
        gw = wn * w
        gh = hn * h
        aw = _per_row([float(_ANCH[i][q][0]) for q in range(3)], a_r)
        ah = _per_row([float(_ANCH[i][q][1]) for q in range(3)], a_r)

        rw = gw / aw
        rh = gh / ah
        anc_ok = jnp.maximum(jnp.maximum(rw, 1.0 / rw),
                             jnp.maximum(rh, 1.0 / rh)) < _ANCHOR_T

        fx = gx - jnp.floor(gx)
        fy = gy - jnp.floor(gy)
        gxi = w - gx
        gyi = h - gy
        fxi = gxi - jnp.floor(gxi)
        fyi = gyi - jnp.floor(gyi)
        c1 = ((fx < 0.5) & (gx > 1.0)).astype(jnp.float32)
        c2 = ((fy < 0.5) & (gy > 1.0)).astype(jnp.float32)
        c3 = ((fxi < 0.5) & (gxi > 1.0)).astype(jnp.float32)
        c4 = ((fyi < 0.5) & (gyi > 1.0)).astype(jnp.float32)
        omode = row // 3
        w0 = (omode == 0).astype(jnp.float32)
        w1 = (omode == 1).astype(jnp.float32)
        w2 = (omode == 2).astype(jnp.float32)
        w3 = (omode == 3).astype(jnp.float32)
        w4 = (omode == 4).astype(jnp.float32)
        omf = w0 + w1 * c1 + w2 * c2 + w3 * c3 + w4 * c4

        mf = (omf * anc_ok.astype(jnp.float32) * validf
              * rowf_ok.astype(jnp.float32))
        m = mf > 0.5

        sx = gx - offx
        sy = gy - offy
        gi = sx.astype(jnp.int32)
        gj = sy.astype(jnp.int32)
        tbx = gx - gi.astype(jnp.float32)
        tby = gy - gj.astype(jnp.float32)

        b_i = bidx.astype(jnp.int32)
        lin = ((b_i * _NA + a_r) * h + gj) * w + gi
        col = lax.broadcasted_iota(jnp.int32, (16, ntp), 1)
        spread = row * ntp + col
        idx_ref[i] = jnp.where(m, lin, spread)
        meta_ref[i, 0] = jnp.broadcast_to(mf, (16, ntp))
        meta_ref[i, 1] = jnp.broadcast_to(tbx, (16, ntp))
        meta_ref[i, 2] = jnp.broadcast_to(tby, (16, ntp))
        meta_ref[i, 3] = jnp.broadcast_to(gw, (16, ntp))
        meta_ref[i, 4] = jnp.broadcast_to(gh, (16, ntp))
        meta_ref[i, 5] = jnp.broadcast_to(cls_t, (16, ntp))
        meta_ref[i, 6] = jnp.broadcast_to(aw, (16, ntp))
        meta_ref[i, 7] = jnp.broadcast_to(ah, (16, ntp))


def _k2_body(p0_ref, p1_ref, p2_ref, q0_ref, q1_ref, q2_ref, dsum_ref):
    k = pl.program_id(0)

    @pl.when(k == 0)
    def _init():
        dsum_ref[...] = jnp.zeros_like(dsum_ref)

    lane = lax.broadcasted_iota(jnp.int32, (1, 128), 1)
    part = []
    for p_ref, q_ref in ((p0_ref, q0_ref), (p1_ref, q1_ref), (p2_ref, q2_ref)):
        x = p_ref[...]
        part.append(jnp.sum(_softplus_terms(x[:, 4:5])))
        pad = jnp.zeros((x.shape[0], 128 - _NCH), jnp.float32)
        q_ref[...] = jnp.concatenate((x, pad), axis=1)
    dsum_ref[...] = dsum_ref[...] + (jnp.where(lane == 0, part[0], 0.0)
                                     + jnp.where(lane == 1, part[1], 0.0)
                                     + jnp.where(lane == 2, part[2], 0.0))


def _sc_gather(idx0, idx1, idx2, q0, q1, q2):
    nwork, nchunk, _ = idx0.shape
    upw = nchunk * 128
    n = nwork * upw
    mesh = plsc.VectorSubcoreMesh(core_axis_name="c", subcore_axis_name="s")
    row_t = jax.ShapeDtypeStruct((n, 128), jnp.float32)

    @functools.partial(
        pl.kernel,
        out_type=(row_t, row_t, row_t),
        mesh=mesh,
        scratch_types=[
            pltpu.VMEM((nchunk, 128), jnp.int32),
            pltpu.VMEM((upw, 128), jnp.float32),
            pltpu.SemaphoreType.DMA,
        ],
    )
    def k(i0, i1, i2, t0, t1, t2, o0, o1, o2, idx_v, rows_v, sem):
        wid = lax.axis_index("s") * 2 + lax.axis_index("c")
        for ih, tab, out in ((i0, t0, o0), (i1, t1, o1), (i2, t2, o2)):
            pltpu.sync_copy(ih.at[wid], idx_v)
            descs = []
            for c in range(nchunk):
                descs.append(pltpu.async_copy(
                    tab.at[idx_v.at[c]],
                    rows_v.at[pl.ds(c * 128, 128), :], sem))
            for d in descs:
                d.wait()
            pltpu.sync_copy(rows_v, out.at[pl.ds(wid * upw, upw)])

    return k(idx0, idx1, idx2, q0, q1, q2)


def _k4_body(ps0_ref, ps1_ref, ps2_ref, meta_ref, dsum_ref, out_ref, acc_ref,
             *, nsteps, npix, bs):
    j = pl.program_id(0)

    @pl.when(j == 0)
    def _init():
        acc_ref[...] = jnp.zeros_like(acc_ref)

    lane = lax.broadcasted_iota(jnp.int32, (1, 128), 1)
    for i, ps_ref in enumerate((ps0_ref, ps1_ref, ps2_ref)):
        ps = ps_ref[...].astype(jnp.float32)
        mt = meta_ref[i]
        m = mt[:, 0:1]
        tbx, tby = mt[:, 1:2], mt[:, 2:3]
        tbw, tbh = mt[:, 3:4], mt[:, 4:5]
        cls_t = mt[:, 5:6]
        aw, ah = mt[:, 6:7], mt[:, 7:8]

        s = _sigmoid(ps[:, 0:4])
        px = s[:, 0:1] * 2.0 - 0.5
        py = s[:, 1:2] * 2.0 - 0.5
        pw = (s[:, 2:3] * 2.0) ** 2 * aw
        ph = (s[:, 3:4] * 2.0) ** 2 * ah

        b1x1, b1x2 = px - pw * 0.5, px + pw * 0.5
        b1y1, b1y2 = py - ph * 0.5, py + ph * 0.5
        b2x1, b2x2 = tbx - tbw * 0.5, tbx + tbw * 0.5
        b2y1, b2y2 = tby - tbh * 0.5, tby + tbh * 0.5
        inter = (jnp.maximum(jnp.minimum(b1x2, b2x2) - jnp.maximum(b1x1, b2x1), 0.0)
                 * jnp.maximum(jnp.minimum(b1y2, b2y2) - jnp.maximum(b1y1, b2y1), 0.0))
        union = pw * ph + tbw * tbh - inter + _EPS
        iou = inter / union
        cw = jnp.maximum(b1x2, b2x2) - jnp.minimum(b1x1, b2x1)
        ch = jnp.maximum(b1y2, b2y2) - jnp.minimum(b1y1, b2y1)
        c2 = cw * cw + ch * ch + _EPS
        rho2 = ((b2x1 + b2x2 - b1x1 - b1x2) ** 2
                + (b2y1 + b2y2 - b1y1 - b1y2) ** 2) * 0.25
        v = ((4.0 / np.pi ** 2)
             * (_atan_pos(tbw / (tbh + _EPS)) - _atan_pos(pw / (ph + _EPS))) ** 2)
        alpha = v / (v - iou + (1.0 + _EPS))
        ciou = iou - (rho2 / c2 + v * alpha)

        lbox_num = jnp.sum((1.0 - ciou) * m)
        x4 = ps[:, 4:5]
        s_obj = jnp.sum(m * x4 * jnp.maximum(ciou, 0.0))
        cnt = jnp.sum(m)

        xs = ps[:, 5:_NCH]
        sp_sum = jnp.sum(_softplus_terms(xs), axis=1, keepdims=True)
        cid = lax.broadcasted_iota(jnp.int32, xs.shape, 1)
        pick = jnp.sum(jnp.where(cid == cls_t.astype(jnp.int32), xs, 0.0),
                       axis=1, keepdims=True)
        cls_num = jnp.sum(m * (sp_sum - pick))

        contrib = (jnp.where(lane == 0, lbox_num, 0.0)
                   + jnp.where(lane == 1, cls_num, 0.0)
                   + jnp.where(lane == 2, s_obj, 0.0)
                   + jnp.where(lane == 3, cnt, 0.0))
        acc_ref[i:i + 1, :] = acc_ref[i:i + 1, :] + contrib

    @pl.when(j == nsteps - 1)
    def _final():
        lbox = jnp.float32(0.0)
        lcls = jnp.float32(0.0)
        lobj = jnp.float32(0.0)
        for i in range(_NL):
            lbox_num = acc_ref[i, 0]
            cls_num = acc_ref[i, 1]
            s_obj = acc_ref[i, 2]
            cnt = acc_ref[i, 3]
            denom = jnp.maximum(cnt, 1.0)
            lbox = lbox + jnp.where(cnt > 0, lbox_num / denom, 0.0)
            lcls = lcls + jnp.where(cnt > 0, cls_num / (denom * _NC), 0.0)
            dense = dsum_ref[0, i]
            lobj = lobj + ((dense - s_obj) / npix[i]) * _BAL[i]
        lbox = lbox * _G_GIOU
        lobj = lobj * _G_OBJ
        lcls = lcls * _G_CLS
        loss = (lbox + lobj + lcls) * bs
        out_ref[0:1, :] = jnp.where(lane == 0, loss, 0.0)
        out_ref[1:2, :] = (jnp.where(lane == 0, lbox, 0.0)
                           + jnp.where(lane == 1, lobj, 0.0)
                           + jnp.where(lane == 2, lcls, 0.0))


def kernel(p0, p1, p2, raw_targets, labels_length, img_size):
    b, mb, _ = raw_targets.shape
    mpad = 64
    ntp = b * mpad
    nupd = 16 * ntp
    hw = tuple((p.shape[2], p.shape[3]) for p in (p0, p1, p2))
    npix = tuple(p.shape[0] * p.shape[1] * p.shape[2] * p.shape[3]
                 for p in (p0, p1, p2))

    rt = jnp.pad(raw_targets, ((0, 0), (0, mpad - mb), (0, 0)))
    isz = jnp.asarray(img_size, jnp.float32)
    validf = (jnp.arange(mpad)[None, :] < labels_length[:, None]).astype(jnp.float32)
    bidxf = jnp.broadcast_to(jnp.arange(b, dtype=jnp.float32)[:, None], (b, mpad))
    tinfo = jnp.stack([
        rt[:, :, 0].reshape(-1),
        (rt[:, :, 1] / isz).reshape(-1),
        (rt[:, :, 2] / isz).reshape(-1),
        (rt[:, :, 3] / isz).reshape(-1),
        (rt[:, :, 4] / isz).reshape(-1),
        validf.reshape(-1),
        bidxf.reshape(-1),
        jnp.zeros((ntp,), jnp.float32),
    ])

    idx, meta = pl.pallas_call(
        functools.partial(_k1_body, hw=hw),
        out_shape=(jax.ShapeDtypeStruct((_NL, 16, ntp), jnp.int32),
                   jax.ShapeDtypeStruct((_NL, 8, 16, ntp), jnp.float32)),
    )(tinfo)

    t0 = p0.reshape(npix[0], _NCH)
    t1 = p1.reshape(npix[1], _NCH)
    t2 = p2.reshape(npix[2], _NCH)
    nsteps = 20
    blk0, blk1, blk2 = npix[0] // nsteps, npix[1] // nsteps, npix[2] // nsteps
    q0, q1, q2, dsum = pl.pallas_call(
        _k2_body,
        grid=(nsteps,),
        in_specs=[
            pl.BlockSpec((blk0, _NCH), lambda k: (k, 0)),
            pl.BlockSpec((blk1, _NCH), lambda k: (k, 0)),
            pl.BlockSpec((blk2, _NCH), lambda k: (k, 0)),
        ],
        out_specs=[
            pl.BlockSpec((blk0, 128), lambda k: (k, 0)),
            pl.BlockSpec((blk1, 128), lambda k: (k, 0)),
            pl.BlockSpec((blk2, 128), lambda k: (k, 0)),
            pl.BlockSpec((1, 128), lambda k: (0, 0)),
        ],
        out_shape=(jax.ShapeDtypeStruct((npix[0], 128), jnp.float32),
                   jax.ShapeDtypeStruct((npix[1], 128), jnp.float32),
                   jax.ShapeDtypeStruct((npix[2], 128), jnp.float32),
                   jax.ShapeDtypeStruct((1, 128), jnp.float32)),
    )(t0, t1, t2)

    idx_w = idx.reshape(_NL, 32, nupd // (32 * 128), 128)
    ps0, ps1, ps2 = _sc_gather(idx_w[0], idx_w[1], idx_w[2], q0, q1, q2)

    meta_u = jnp.transpose(meta.reshape(_NL, 8, nupd), (0, 2, 1))
    blk = 2048
    nblk = nupd // blk
    out = pl.pallas_call(
        functools.partial(_k4_body, nsteps=nblk, npix=npix, bs=float(b)),
        grid=(nblk,),
        in_specs=[
            pl.BlockSpec((blk, 128), lambda j: (j, 0)),
            pl.BlockSpec((blk, 128), lambda j: (j, 0)),
            pl.BlockSpec((blk, 128), lambda j: (j, 0)),
            pl.BlockSpec((_NL, blk, 8), lambda j: (0, j, 0)),
            pl.BlockSpec((1, 128), lambda j: (0, 0)),
        ],
        out_specs=pl.BlockSpec((8, 128), lambda j: (0, 0)),
        out_shape=jax.ShapeDtypeStruct((8, 128), jnp.float32),
        scratch_shapes=[pltpu.VMEM((8, 128), jnp.float32)],
    )(ps0, ps1, ps2, meta_u, dsum)

    loss = out[0, 0:1]
    stack = out[1, 0:3]
    return (loss, stack)


# speedup vs baseline: 4.9788x; 1.0008x over previous
import functools

import numpy as np
import jax
import jax.numpy as jnp
from jax import lax
from jax.experimental import pallas as pl
from jax.experimental.pallas import tpu as pltpu
from jax.experimental.pallas import tpu_sc as plsc

_NL = 3
_NA = 3
_NC = 80
_NCH = _NC + 5
_STRIDES = (8, 16, 32)
_ANCH = (np.array([[[10.0, 13.0], [16.0, 30.0], [33.0, 23.0]],
                   [[30.0, 61.0], [62.0, 45.0], [59.0, 119.0]],
                   [[116.0, 90.0], [156.0, 198.0], [373.0, 326.0]]], np.float32)
         / np.array(_STRIDES, np.float32)[:, None, None])
_BAL = (4.0, 1.0, 0.4)
_G_GIOU, _G_OBJ, _G_CLS = 0.05, 1.0, 0.5
_ANCHOR_T = 4.0
_EPS = 1e-7
_OFF = np.array([[0.0, 0.0], [0.5, 0.0], [0.0, 0.5], [-0.5, 0.0], [0.0, -0.5]],
                np.float32)


def _sigmoid(x):
    return 1.0 / (1.0 + jnp.exp(-x))


def _softplus_terms(x):
    return jnp.maximum(x, 0.0) + jnp.log(1.0 + jnp.exp(-jnp.abs(x)))


def _atan_pos(x):
    inv = x > 1.0
    z = jnp.where(inv, 1.0 / jnp.maximum(x, 1e-30), x)
    t = z / (1.0 + jnp.sqrt(1.0 + z * z))
    t2 = t * t
    p = t * (1.0 + t2 * (-1.0 / 3.0 + t2 * (0.2 + t2 * (-1.0 / 7.0
             + t2 * (1.0 / 9.0 - t2 / 11.0)))))
    a = 2.0 * p
    return jnp.where(inv, (np.pi / 2.0) - a, a)


def _k1_body(tinfo_ref, idx_ref, meta_ref, *, hw):
    ntp = tinfo_ref.shape[1]
    cls_t = tinfo_ref[0:1, :]
    xn = tinfo_ref[1:2, :]
    yn = tinfo_ref[2:3, :]
    wn = tinfo_ref[3:4, :]
    hn = tinfo_ref[4:5, :]
    validf = tinfo_ref[5:6, :]
    bidx = tinfo_ref[6:7, :]

    row = lax.broadcasted_iota(jnp.int32, (16, 1), 0)
    a_r = row % 3
    rowf_ok = (row < 15)
    omode0 = jnp.minimum(row // 3, 4)
    zero = jnp.zeros_like(omode0, jnp.float32)

    def _per_row(vals, sel):
        out = zero + vals[0]
        for q in range(1, len(vals)):
            out = jnp.where(sel == q, vals[q], out)
        return out

    offx = _per_row([_OFF[o][0] for o in range(5)], omode0)
    offy = _per_row([_OFF[o][1] for o in range(5)], omode0)

    for i in range(_NL):
        h, w = hw[i]
        gx = xn * w
        gy = yn * h
        gw = wn * w
        gh = hn * h
        aw = _per_row([float(_ANCH[i][q][0]) for q in range(3)], a_r)
        ah = _per_row([float(_ANCH[i][q][1]) for q in range(3)], a_r)

        rw = gw / aw
        rh = gh / ah
        anc_ok = jnp.maximum(jnp.maximum(rw, 1.0 / rw),
                             jnp.maximum(rh, 1.0 / rh)) < _ANCHOR_T

        fx = gx - jnp.floor(gx)
        fy = gy - jnp.floor(gy)
        gxi = w - gx
        gyi = h - gy
        fxi = gxi - jnp.floor(gxi)
        fyi = gyi - jnp.floor(gyi)
        c1 = ((fx < 0.5) & (gx > 1.0)).astype(jnp.float32)
        c2 = ((fy < 0.5) & (gy > 1.0)).astype(jnp.float32)
        c3 = ((fxi < 0.5) & (gxi > 1.0)).astype(jnp.float32)
        c4 = ((fyi < 0.5) & (gyi > 1.0)).astype(jnp.float32)
        omode = row // 3
        w0 = (omode == 0).astype(jnp.float32)
        w1 = (omode == 1).astype(jnp.float32)
        w2 = (omode == 2).astype(jnp.float32)
        w3 = (omode == 3).astype(jnp.float32)
        w4 = (omode == 4).astype(jnp.float32)
        omf = w0 + w1 * c1 + w2 * c2 + w3 * c3 + w4 * c4

        mf = (omf * anc_ok.astype(jnp.float32) * validf
              * rowf_ok.astype(jnp.float32))
        m = mf > 0.5

        sx = gx - offx
        sy = gy - offy
        gi = sx.astype(jnp.int32)
        gj = sy.astype(jnp.int32)
        tbx = gx - gi.astype(jnp.float32)
        tby = gy - gj.astype(jnp.float32)

        b_i = bidx.astype(jnp.int32)
        lin = ((b_i * _NA + a_r) * h + gj) * w + gi
        col = lax.broadcasted_iota(jnp.int32, (16, ntp), 1)
        spread = row * ntp + col
        idx_ref[i] = jnp.where(m, lin, spread)
        meta_ref[i, 0] = jnp.broadcast_to(mf, (16, ntp))
        meta_ref[i, 1] = jnp.broadcast_to(tbx, (16, ntp))
        meta_ref[i, 2] = jnp.broadcast_to(tby, (16, ntp))
        meta_ref[i, 3] = jnp.broadcast_to(gw, (16, ntp))
        meta_ref[i, 4] = jnp.broadcast_to(gh, (16, ntp))
        meta_ref[i, 5] = jnp.broadcast_to(cls_t, (16, ntp))
        meta_ref[i, 6] = jnp.broadcast_to(aw, (16, ntp))
        meta_ref[i, 7] = jnp.broadcast_to(ah, (16, ntp))


def _k2_body(p0_ref, p1_ref, p2_ref, q0_ref, q1_ref, q2_ref, dsum_ref):
    k = pl.program_id(0)

    @pl.when(k == 0)
    def _init():
        dsum_ref[...] = jnp.zeros_like(dsum_ref)

    lane = lax.broadcasted_iota(jnp.int32, (1, 128), 1)
    part = []
    for p_ref, q_ref in ((p0_ref, q0_ref), (p1_ref, q1_ref), (p2_ref, q2_ref)):
        x = p_ref[...]
        part.append(jnp.sum(_softplus_terms(x[:, 4:5])))
        pad = jnp.zeros((x.shape[0], 128 - _NCH), jnp.float32)
        q_ref[...] = jnp.concatenate((x, pad), axis=1)
    dsum_ref[...] = dsum_ref[...] + (jnp.where(lane == 0, part[0], 0.0)
                                     + jnp.where(lane == 1, part[1], 0.0)
                                     + jnp.where(lane == 2, part[2], 0.0))


def _sc_gather(idx0, idx1, idx2, q0, q1, q2):
    nwork, nchunk, _ = idx0.shape
    upw = nchunk * 128
    n = nwork * upw
    mesh = plsc.VectorSubcoreMesh(core_axis_name="c", subcore_axis_name="s")
    row_t = jax.ShapeDtypeStruct((n, 128), jnp.float32)

    @functools.partial(
        pl.kernel,
        out_type=(row_t, row_t, row_t),
        mesh=mesh,
        scratch_types=[
            pltpu.VMEM((nchunk, 128), jnp.int32),
            pltpu.VMEM((upw, 128), jnp.float32),
            pltpu.SemaphoreType.DMA,
        ],
    )
    def k(i0, i1, i2, t0, t1, t2, o0, o1, o2, idx_v, rows_v, sem):
        wid = lax.axis_index("s") * 2 + lax.axis_index("c")
        for ih, tab, out in ((i0, t0, o0), (i1, t1, o1), (i2, t2, o2)):
            pltpu.sync_copy(ih.at[wid], idx_v)
            descs = []
            for c in range(nchunk):
                descs.append(pltpu.async_copy(
                    tab.at[idx_v.at[c]],
                    rows_v.at[pl.ds(c * 128, 128), :], sem))
            for d in descs:
                d.wait()
            pltpu.sync_copy(rows_v, out.at[pl.ds(wid * upw, upw)])

    return k(idx0, idx1, idx2, q0, q1, q2)


def _k4_body(ps0_ref, ps1_ref, ps2_ref, meta_ref, dsum_ref, out_ref, acc_ref,
             *, nsteps, npix, bs):
    j = pl.program_id(0)

    @pl.when(j == 0)
    def _init():
        acc_ref[...] = jnp.zeros_like(acc_ref)

    lane = lax.broadcasted_iota(jnp.int32, (1, 128), 1)
    for i, ps_ref in enumerate((ps0_ref, ps1_ref, ps2_ref)):
        ps = ps_ref[...].astype(jnp.float32)
        mt = meta_ref[i]
        m = mt[:, 0:1]
        tbx, tby = mt[:, 1:2], mt[:, 2:3]
        tbw, tbh = mt[:, 3:4], mt[:, 4:5]
        cls_t = mt[:, 5:6]
        aw, ah = mt[:, 6:7], mt[:, 7:8]

        s = _sigmoid(ps[:, 0:4])
        px = s[:, 0:1] * 2.0 - 0.5
        py = s[:, 1:2] * 2.0 - 0.5
        pw = (s[:, 2:3] * 2.0) ** 2 * aw
        ph = (s[:, 3:4] * 2.0) ** 2 * ah

        b1x1, b1x2 = px - pw * 0.5, px + pw * 0.5
        b1y1, b1y2 = py - ph * 0.5, py + ph * 0.5
        b2x1, b2x2 = tbx - tbw * 0.5, tbx + tbw * 0.5
        b2y1, b2y2 = tby - tbh * 0.5, tby + tbh * 0.5
        inter = (jnp.maximum(jnp.minimum(b1x2, b2x2) - jnp.maximum(b1x1, b2x1), 0.0)
                 * jnp.maximum(jnp.minimum(b1y2, b2y2) - jnp.maximum(b1y1, b2y1), 0.0))
        union = pw * ph + tbw * tbh - inter + _EPS
        iou = inter / union
        cw = jnp.maximum(b1x2, b2x2) - jnp.minimum(b1x1, b2x1)
        ch = jnp.maximum(b1y2, b2y2) - jnp.minimum(b1y1, b2y1)
        c2 = cw * cw + ch * ch + _EPS
        rho2 = ((b2x1 + b2x2 - b1x1 - b1x2) ** 2
                + (b2y1 + b2y2 - b1y1 - b1y2) ** 2) * 0.25
        v = ((4.0 / np.pi ** 2)
             * (_atan_pos(tbw / (tbh + _EPS)) - _atan_pos(pw / (ph + _EPS))) ** 2)
        alpha = v / (v - iou + (1.0 + _EPS))
        ciou = iou - (rho2 / c2 + v * alpha)

        lbox_num = jnp.sum((1.0 - ciou) * m)
        x4 = ps[:, 4:5]
        s_obj = jnp.sum(m * x4 * jnp.maximum(ciou, 0.0))
        cnt = jnp.sum(m)

        xs = ps[:, 5:_NCH]
        sp_sum = jnp.sum(_softplus_terms(xs), axis=1, keepdims=True)
        cid = lax.broadcasted_iota(jnp.int32, xs.shape, 1)
        pick = jnp.sum(jnp.where(cid == cls_t.astype(jnp.int32), xs, 0.0),
                       axis=1, keepdims=True)
        cls_num = jnp.sum(m * (sp_sum - pick))

        contrib = (jnp.where(lane == 0, lbox_num, 0.0)
                   + jnp.where(lane == 1, cls_num, 0.0)
                   + jnp.where(lane == 2, s_obj, 0.0)
                   + jnp.where(lane == 3, cnt, 0.0))
        acc_ref[i:i + 1, :] = acc_ref[i:i + 1, :] + contrib

    @pl.when(j == nsteps - 1)
    def _final():
        lbox = jnp.float32(0.0)
        lcls = jnp.float32(0.0)
        lobj = jnp.float32(0.0)
        for i in range(_NL):
            lbox_num = acc_ref[i, 0]
            cls_num = acc_ref[i, 1]
            s_obj = acc_ref[i, 2]
            cnt = acc_ref[i, 3]
            denom = jnp.maximum(cnt, 1.0)
            lbox = lbox + jnp.where(cnt > 0, lbox_num / denom, 0.0)
            lcls = lcls + jnp.where(cnt > 0, cls_num / (denom * _NC), 0.0)
            dense = dsum_ref[0, i]
            lobj = lobj + ((dense - s_obj) / npix[i]) * _BAL[i]
        lbox = lbox * _G_GIOU
        lobj = lobj * _G_OBJ
        lcls = lcls * _G_CLS
        loss = (lbox + lobj + lcls) * bs
        out_ref[0:1, :] = jnp.where(lane == 0, loss, 0.0)
        out_ref[1:2, :] = (jnp.where(lane == 0, lbox, 0.0)
                           + jnp.where(lane == 1, lobj, 0.0)
                           + jnp.where(lane == 2, lcls, 0.0))


def kernel(p0, p1, p2, raw_targets, labels_length, img_size):
    b, mb, _ = raw_targets.shape
    mpad = 64
    ntp = b * mpad
    nupd = 16 * ntp
    hw = tuple((p.shape[2], p.shape[3]) for p in (p0, p1, p2))
    npix = tuple(p.shape[0] * p.shape[1] * p.shape[2] * p.shape[3]
                 for p in (p0, p1, p2))

    rt = jnp.pad(raw_targets, ((0, 0), (0, mpad - mb), (0, 0)))
    isz = jnp.asarray(img_size, jnp.float32)
    validf = (jnp.arange(mpad)[None, :] < labels_length[:, None]).astype(jnp.float32)
    bidxf = jnp.broadcast_to(jnp.arange(b, dtype=jnp.float32)[:, None], (b, mpad))
    tinfo = jnp.stack([
        rt[:, :, 0].reshape(-1),
        (rt[:, :, 1] / isz).reshape(-1),
        (rt[:, :, 2] / isz).reshape(-1),
        (rt[:, :, 3] / isz).reshape(-1),
        (rt[:, :, 4] / isz).reshape(-1),
        validf.reshape(-1),
        bidxf.reshape(-1),
        jnp.zeros((ntp,), jnp.float32),
    ])

    idx, meta = pl.pallas_call(
        functools.partial(_k1_body, hw=hw),
        out_shape=(jax.ShapeDtypeStruct((_NL, 16, ntp), jnp.int32),
                   jax.ShapeDtypeStruct((_NL, 8, 16, ntp), jnp.float32)),
    )(tinfo)

    t0 = p0.reshape(npix[0], _NCH)
    t1 = p1.reshape(npix[1], _NCH)
    t2 = p2.reshape(npix[2], _NCH)
    nsteps = 16
    blk0, blk1, blk2 = npix[0] // nsteps, npix[1] // nsteps, npix[2] // nsteps
    q0, q1, q2, dsum = pl.pallas_call(
        _k2_body,
        grid=(nsteps,),
        in_specs=[
            pl.BlockSpec((blk0, _NCH), lambda k: (k, 0)),
            pl.BlockSpec((blk1, _NCH), lambda k: (k, 0)),
            pl.BlockSpec((blk2, _NCH), lambda k: (k, 0)),
        ],
        out_specs=[
            pl.BlockSpec((blk0, 128), lambda k: (k, 0)),
            pl.BlockSpec((blk1, 128), lambda k: (k, 0)),
            pl.BlockSpec((blk2, 128), lambda k: (k, 0)),
            pl.BlockSpec((1, 128), lambda k: (0, 0)),
        ],
        out_shape=(jax.ShapeDtypeStruct((npix[0], 128), jnp.float32),
                   jax.ShapeDtypeStruct((npix[1], 128), jnp.float32),
                   jax.ShapeDtypeStruct((npix[2], 128), jnp.float32),
                   jax.ShapeDtypeStruct((1, 128), jnp.float32)),
    )(t0, t1, t2)

    idx_w = idx.reshape(_NL, 32, nupd // (32 * 128), 128)
    ps0, ps1, ps2 = _sc_gather(idx_w[0], idx_w[1], idx_w[2], q0, q1, q2)

    meta_u = jnp.transpose(meta.reshape(_NL, 8, nupd), (0, 2, 1))
    blk = 2048
    nblk = nupd // blk
    out = pl.pallas_call(
        functools.partial(_k4_body, nsteps=nblk, npix=npix, bs=float(b)),
        grid=(nblk,),
        in_specs=[
            pl.BlockSpec((blk, 128), lambda j: (j, 0)),
            pl.BlockSpec((blk, 128), lambda j: (j, 0)),
            pl.BlockSpec((blk, 128), lambda j: (j, 0)),
            pl.BlockSpec((_NL, blk, 8), lambda j: (0, j, 0)),
            pl.BlockSpec((1, 128), lambda j: (0, 0)),
        ],
        out_specs=pl.BlockSpec((8, 128), lambda j: (0, 0)),
        out_shape=jax.ShapeDtypeStruct((8, 128), jnp.float32),
        scratch_shapes=[pltpu.VMEM((8, 128), jnp.float32)],
    )(ps0, ps1, ps2, meta_u, dsum)

    loss = out[0, 0:1]
    stack = out[1, 0:3]
    return (loss, stack)
